# Initial kernel scaffold; baseline (speedup 1.0000x reference)
import jax
import jax.numpy as jnp
from jax.experimental import pallas as pl


def kernel(x, edge_index, edge_attr, W1, b1, W2, b2, Wc, bc):
    raise NotImplementedError("write your pallas kernel here")


# speedup vs baseline: 8.4876x; 8.4876x over previous
import functools

import jax
import jax.numpy as jnp
from jax import lax
from jax.experimental import pallas as pl
from jax.experimental.pallas import tpu as pltpu
from jax.experimental.pallas import tpu_sc as plsc

NC = 2
NS = 16
NW = NC * NS
CH = 128
LANES = 16


def _make_deg(n, nch, epw):
    dpt = -(-n // (NS * CH)) * CH
    n1 = dpt * NS
    mesh = plsc.VectorSubcoreMesh(core_axis_name="c", subcore_axis_name="s")

    @functools.partial(
        pl.kernel,
        out_type=jax.ShapeDtypeStruct((NC, n1), jnp.float32),
        mesh=mesh,
        scratch_types=[
            pltpu.VMEM_SHARED((n1,), jnp.float32),
            pltpu.VMEM((dpt,), jnp.float32),
            pltpu.VMEM((CH,), jnp.int32),
            pltpu.VMEM((CH,), jnp.float32),
        ],
    )
    def deg_kernel(dst_hbm, ew_hbm, out_hbm, acc_sh, zero_v, didx_v, ew_v):
        c = lax.axis_index("c")
        s = lax.axis_index("s")
        wid = s * NC + c
        for j in range(dpt // LANES):
            zero_v[pl.ds(j * LANES, LANES)] = jnp.zeros((LANES,), jnp.float32)
        pltpu.sync_copy(zero_v, acc_sh.at[pl.ds(s * dpt, dpt)])
        plsc.subcore_barrier()
        base = wid * epw

        def chunk(i, carry):
            b = base + i * CH
            pltpu.sync_copy(dst_hbm.at[pl.ds(b, CH)], didx_v)
            pltpu.sync_copy(ew_hbm.at[pl.ds(b, CH)], ew_v)
            pltpu.sync_copy(ew_v, acc_sh.at[didx_v], add=True)
            return carry

        lax.fori_loop(0, nch, chunk, 0)
        plsc.subcore_barrier()
        pltpu.sync_copy(acc_sh.at[pl.ds(s * dpt, dpt)],
                        out_hbm.at[c, pl.ds(s * dpt, dpt)])

    return deg_kernel, n1


def _make_edge(n, nch, epw, d):
    rpt = -(-n // (NS * CH)) * CH
    np_ = rpt * NS
    mesh = plsc.VectorSubcoreMesh(core_axis_name="c", subcore_axis_name="s")

    @functools.partial(
        pl.kernel,
        out_type=jax.ShapeDtypeStruct((NC, np_, d), jnp.float32),
        mesh=mesh,
        scratch_types=[
            pltpu.VMEM_SHARED((np_, d), jnp.float32),
            pltpu.VMEM((CH,), jnp.int32),
            pltpu.VMEM((CH,), jnp.int32),
            pltpu.VMEM((CH,), jnp.float32),
            pltpu.VMEM((CH, d), jnp.float32),
            pltpu.SemaphoreType.DMA,
        ],
        compiler_params=pltpu.CompilerParams(use_tc_tiling_on_sc=False),
    )
    def edge_kernel(src_hbm, dst_hbm, ew_hbm, g_hbm, out_hbm,
                    acc_sh, sidx_v, didx_v, ew_v, rows_v, sem):
        c = lax.axis_index("c")
        s = lax.axis_index("s")
        wid = s * NC + c

        def zrow(k, carry):
            for j in range(d // LANES):
                rows_v[k, pl.ds(j * LANES, LANES)] = jnp.zeros((LANES,),
                                                               jnp.float32)
            return carry

        lax.fori_loop(0, CH, zrow, 0)
        for t in range(rpt // CH):
            pltpu.sync_copy(rows_v, acc_sh.at[pl.ds(s * rpt + t * CH, CH)])
        plsc.subcore_barrier()
        base = wid * epw

        def chunk(i, carry):
            b = base + i * CH
            pltpu.sync_copy(src_hbm.at[pl.ds(b, CH)], sidx_v)
            pltpu.sync_copy(dst_hbm.at[pl.ds(b, CH)], didx_v)
            pltpu.sync_copy(ew_hbm.at[pl.ds(b, CH)], ew_v)
            pltpu.async_copy(g_hbm.at[sidx_v], rows_v, sem).wait()

            def scale(kb, carry2):
                ew16 = ew_v[pl.ds(kb * LANES, LANES)]
                for k in range(LANES):
                    w = jnp.full((LANES,), ew16[k], jnp.float32)
                    for j in range(d // LANES):
                        sl = pl.ds(j * LANES, LANES)
                        rows_v[kb * LANES + k, sl] = rows_v[kb * LANES + k, sl] * w
                return carry2

            lax.fori_loop(0, CH // LANES, scale, 0)
            pltpu.sync_copy(rows_v, acc_sh.at[didx_v], add=True)
            return carry

        lax.fori_loop(0, nch, chunk, 0)
        plsc.subcore_barrier()
        for t in range(rpt // CH):
            r0 = s * rpt + t * CH
            pltpu.sync_copy(acc_sh.at[pl.ds(r0, CH)], out_hbm.at[c, pl.ds(r0, CH)])

    return edge_kernel


def _tc_pre(d0, d1, x, w1):
    n, din = x.shape
    h1 = w1.shape[1]
    bn = 1000

    def body(d0_r, d1_r, x_r, w_r, g_r, dinv_r):
        dinv = lax.rsqrt(d0_r[...] + d1_r[...] + 1.0)
        dinv_r[...] = dinv
        g_r[...] = jnp.dot(x_r[...], w_r[...],
                           preferred_element_type=jnp.float32) * dinv

    return pl.pallas_call(
        body,
        grid=(n // bn,),
        in_specs=[
            pl.BlockSpec((bn, 1), lambda i: (i, 0)),
            pl.BlockSpec((bn, 1), lambda i: (i, 0)),
            pl.BlockSpec((bn, din), lambda i: (i, 0)),
            pl.BlockSpec((din, h1), lambda i: (0, 0)),
        ],
        out_specs=[
            pl.BlockSpec((bn, h1), lambda i: (i, 0)),
            pl.BlockSpec((bn, 1), lambda i: (i, 0)),
        ],
        out_shape=[
            jax.ShapeDtypeStruct((n, h1), jnp.float32),
            jax.ShapeDtypeStruct((n, 1), jnp.float32),
        ],
    )(d0, d1, x, w1)


def _tc_mid(pa, pb, g1, dinv, b1, w2):
    n, h1 = g1.shape
    h2 = w2.shape[1]
    bn = 1000

    def body(pa_r, pb_r, g_r, dv_r, b_r, w_r, o_r):
        h = jnp.maximum((pa_r[...] + pb_r[...] + g_r[...]) * dv_r[...] + b_r[...],
                        0.0)
        o_r[...] = jnp.dot(h, w_r[...],
                           preferred_element_type=jnp.float32) * dv_r[...]

    return pl.pallas_call(
        body,
        grid=(n // bn,),
        in_specs=[
            pl.BlockSpec((bn, h1), lambda i: (i, 0)),
            pl.BlockSpec((bn, h1), lambda i: (i, 0)),
            pl.BlockSpec((bn, h1), lambda i: (i, 0)),
            pl.BlockSpec((bn, 1), lambda i: (i, 0)),
            pl.BlockSpec((1, h1), lambda i: (0, 0)),
            pl.BlockSpec((h1, h2), lambda i: (0, 0)),
        ],
        out_specs=pl.BlockSpec((bn, h2), lambda i: (i, 0)),
        out_shape=jax.ShapeDtypeStruct((n, h2), jnp.float32),
    )(pa, pb, g1, dinv, b1, w2)


def _tc_out(pa, pb, g2, dinv, b2, wc, bc):
    n, h2 = g2.shape
    cdim = wc.shape[1]
    bn = 1000

    def body(pa_r, pb_r, g_r, dv_r, b_r, w_r, bc_r, o_r):
        o2 = (pa_r[...] + pb_r[...] + g_r[...]) * dv_r[...] + b_r[...]
        o_r[...] = jnp.dot(o2, w_r[...],
                           preferred_element_type=jnp.float32) + bc_r[...]

    return pl.pallas_call(
        body,
        grid=(n // bn,),
        in_specs=[
            pl.BlockSpec((bn, h2), lambda i: (i, 0)),
            pl.BlockSpec((bn, h2), lambda i: (i, 0)),
            pl.BlockSpec((bn, h2), lambda i: (i, 0)),
            pl.BlockSpec((bn, 1), lambda i: (i, 0)),
            pl.BlockSpec((1, h2), lambda i: (0, 0)),
            pl.BlockSpec((h2, cdim), lambda i: (0, 0)),
            pl.BlockSpec((1, cdim), lambda i: (0, 0)),
        ],
        out_specs=pl.BlockSpec((bn, cdim), lambda i: (i, 0)),
        out_shape=jax.ShapeDtypeStruct((n, cdim), jnp.float32),
    )(pa, pb, g2, dinv, b2, wc, bc)


def kernel(x, edge_index, edge_attr, W1, b1, W2, b2, Wc, bc):
    n = x.shape[0]
    e = edge_attr.shape[0]
    src = edge_index[0]
    dst = edge_index[1]

    epc = NW * CH
    ep = -(-e // epc) * epc
    pad = ep - e
    if pad:
        src = jnp.concatenate([src, jnp.zeros((pad,), src.dtype)])
        dst = jnp.concatenate([dst, jnp.zeros((pad,), dst.dtype)])
        ew = jnp.concatenate([edge_attr, jnp.zeros((pad,), edge_attr.dtype)])
    else:
        ew = edge_attr
    epw = ep // NW
    nch = epw // CH

    deg_kernel, _ = _make_deg(n, nch, epw)
    degp = deg_kernel(dst, ew)
    d0 = degp[0, :n, None]
    d1 = degp[1, :n, None]

    g1, dinv = _tc_pre(d0, d1, x, W1)
    p1 = _make_edge(n, nch, epw, W1.shape[1])(src, dst, ew, g1)
    g2 = _tc_mid(p1[0, :n], p1[1, :n], g1, dinv, b1[None, :], W2)
    p2 = _make_edge(n, nch, epw, W2.shape[1])(src, dst, ew, g2)
    return _tc_out(p2[0, :n], p2[1, :n], g2, dinv, b2[None, :], Wc, bc[None, :])
